# speedup vs baseline: 2.9973x; 2.9973x over previous
import functools

import jax
import jax.numpy as jnp
from jax import lax
from jax.experimental import pallas as pl
from jax.experimental.pallas import tpu as pltpu
from jax.experimental.pallas import tpu_sc as plsc

VOCAB = 100000
HIDDEN = 128
B = 4096
S = 200
CHUNK = 100
NCHUNKS = (B * S) // CHUNK
NLANE = 16
NCOL = HIDDEN // NLANE


def _build(num_workers):
    ch_per_w = NCHUNKS // num_workers
    mesh = plsc.VectorSubcoreMesh(core_axis_name="c", subcore_axis_name="s")

    @functools.partial(
        pl.kernel,
        mesh=mesh,
        out_type=jax.ShapeDtypeStruct((NCHUNKS, CHUNK, HIDDEN), jnp.float32),
        scratch_types=[
            pltpu.VMEM((CHUNK,), jnp.int32),
            pltpu.VMEM((CHUNK, HIDDEN), jnp.float32),
            pltpu.VMEM((S, HIDDEN), jnp.float32),
            pltpu.SemaphoreType.DMA,
        ],
    )
    def emb(ids_hbm, word_hbm, pos_hbm, out_hbm, idx_v, rows_v, pos_v, sem):
        cid = lax.axis_index("c")
        sid = lax.axis_index("s")
        wid = sid * 2 + cid
        base = wid * ch_per_w

        pltpu.sync_copy(pos_hbm.at[pl.ds(0, S)], pos_v)

        def step(g, carry):
            c = base + g
            pltpu.sync_copy(ids_hbm.at[c], idx_v)
            pltpu.async_copy(word_hbm.at[idx_v], rows_v, sem).wait()
            start = lax.rem(c, 2) * CHUNK

            def add_row(i, carry2):
                p = start + i
                for k in range(NCOL):
                    sl = pl.ds(k * NLANE, NLANE)
                    rows_v[i, sl] = rows_v[i, sl] + pos_v[p, sl]
                return carry2

            lax.fori_loop(0, CHUNK, add_row, 0)
            pltpu.sync_copy(rows_v, out_hbm.at[c])
            return carry

        lax.fori_loop(0, ch_per_w, step, 0)

    return emb


_emb_kernel = _build(32)


def kernel(input_ids, word_emb, pos_emb):
    ids = input_ids.reshape(NCHUNKS, CHUNK).astype(jnp.int32)
    out = _emb_kernel(ids, word_emb, pos_emb)
    return out.reshape(B, S, HIDDEN)


# speedup vs baseline: 7.5593x; 2.5220x over previous
import functools

import jax
import jax.numpy as jnp
from jax import lax
from jax.experimental import pallas as pl
from jax.experimental.pallas import tpu as pltpu
from jax.experimental.pallas import tpu_sc as plsc

VOCAB = 100000
HIDDEN = 128
B = 4096
S = 200
CHUNK = 100
NCHUNKS = (B * S) // CHUNK
NLANE = 16
NCOL = HIDDEN // NLANE


def _build(num_workers):
    n = NCHUNKS // num_workers
    mesh = plsc.VectorSubcoreMesh(core_axis_name="c", subcore_axis_name="s")

    @functools.partial(
        pl.kernel,
        mesh=mesh,
        out_type=jax.ShapeDtypeStruct((NCHUNKS, CHUNK, HIDDEN), jnp.float32),
        scratch_types=[
            pltpu.VMEM((CHUNK,), jnp.int32),
            pltpu.VMEM((CHUNK,), jnp.int32),
            pltpu.VMEM((CHUNK, HIDDEN), jnp.float32),
            pltpu.VMEM((CHUNK, HIDDEN), jnp.float32),
            pltpu.VMEM((S, HIDDEN), jnp.float32),
            pltpu.SemaphoreType.DMA,
            pltpu.SemaphoreType.DMA,
            pltpu.SemaphoreType.DMA,
            pltpu.SemaphoreType.DMA,
            pltpu.SemaphoreType.DMA,
            pltpu.SemaphoreType.DMA,
        ],
    )
    def emb(ids_hbm, word_hbm, pos_hbm, out_hbm,
            idx0, idx1, rows0, rows1, pos_v,
            gsem0, gsem1, ssem0, ssem1, isem0, isem1):
        idx = (idx0, idx1)
        rows = (rows0, rows1)
        gsem = (gsem0, gsem1)
        ssem = (ssem0, ssem1)
        isem = (isem0, isem1)

        cid = lax.axis_index("c")
        sid = lax.axis_index("s")
        wid = sid * 2 + cid
        base = wid * n

        pltpu.sync_copy(pos_hbm.at[pl.ds(0, S)], pos_v)

        pltpu.sync_copy(ids_hbm.at[base], idx[0])
        pltpu.async_copy(word_hbm.at[idx[0]], rows[0], gsem[0])

        def do_add(b, rows_b):
            start = b * CHUNK

            def add_row(i, carry):
                p = start + i
                for k in range(NCOL):
                    sl = pl.ds(k * NLANE, NLANE)
                    plsc.addupdate(rows_b.at[i, sl], pos_v[p, sl])
                return carry

            lax.fori_loop(0, CHUNK, add_row, 0)

        def body(g, b):
            nb = 1 - b
            c = base + g

            @pl.when(g + 1 < n)
            def _():
                pltpu.async_copy(ids_hbm.at[c + 1], idx[nb], isem[nb])

            pltpu.make_async_copy(word_hbm.at[idx[b]], rows[b], gsem[b]).wait()

            @pl.when((g + 1 < n) & (g > 0))
            def _():
                pltpu.make_async_copy(rows[nb], out_hbm.at[c - 1],
                                      ssem[nb]).wait()

            @pl.when(g + 1 < n)
            def _():
                pltpu.make_async_copy(ids_hbm.at[c + 1], idx[nb],
                                      isem[nb]).wait()
                pltpu.async_copy(word_hbm.at[idx[nb]], rows[nb], gsem[nb])

            do_add(b, rows[b])
            pltpu.async_copy(rows[b], out_hbm.at[c], ssem[b])

        def pair(t, carry):
            body(2 * t, 0)
            body(2 * t + 1, 1)
            return carry

        lax.fori_loop(0, n // 2, pair, 0)

        pltpu.make_async_copy(rows[0], out_hbm.at[base + n - 2], ssem[0]).wait()
        pltpu.make_async_copy(rows[1], out_hbm.at[base + n - 1], ssem[1]).wait()

    return emb


_emb_kernel = _build(32)


def kernel(input_ids, word_emb, pos_emb):
    ids = input_ids.reshape(NCHUNKS, CHUNK).astype(jnp.int32)
    out = _emb_kernel(ids, word_emb, pos_emb)
    return out.reshape(B, S, HIDDEN)


# speedup vs baseline: 7.6550x; 1.0127x over previous
import functools

import jax
import jax.numpy as jnp
from jax import lax
from jax.experimental import pallas as pl
from jax.experimental.pallas import tpu as pltpu
from jax.experimental.pallas import tpu_sc as plsc

VOCAB = 100000
HIDDEN = 128
B = 4096
S = 200
CHUNK = 100
NCHUNKS = (B * S) // CHUNK
NLANE = 16
NCOL = HIDDEN // NLANE
NBUF = 4
LOOK = 2


def _build(num_workers):
    n = NCHUNKS // num_workers
    mesh = plsc.VectorSubcoreMesh(core_axis_name="c", subcore_axis_name="s")

    @functools.partial(
        pl.kernel,
        mesh=mesh,
        out_type=jax.ShapeDtypeStruct((NCHUNKS, CHUNK, HIDDEN), jnp.float32),
        scratch_types=(
            [pltpu.VMEM((CHUNK,), jnp.int32) for _ in range(NBUF)]
            + [pltpu.VMEM((CHUNK, HIDDEN), jnp.float32) for _ in range(NBUF)]
            + [pltpu.VMEM((S, HIDDEN), jnp.float32)]
            + [pltpu.SemaphoreType.DMA for _ in range(3 * NBUF)]
        ),
    )
    def emb(ids_hbm, word_hbm, pos_hbm, out_hbm, *scratch):
        idx = scratch[:NBUF]
        rows = scratch[NBUF:2 * NBUF]
        pos_v = scratch[2 * NBUF]
        gsem = scratch[2 * NBUF + 1:2 * NBUF + 1 + NBUF]
        ssem = scratch[2 * NBUF + 1 + NBUF:2 * NBUF + 1 + 2 * NBUF]
        isem = scratch[2 * NBUF + 1 + 2 * NBUF:]

        cid = lax.axis_index("c")
        sid = lax.axis_index("s")
        wid = sid * 2 + cid
        base = wid * n

        pltpu.sync_copy(pos_hbm.at[pl.ds(0, S)], pos_v)

        for j in range(LOOK):
            pltpu.sync_copy(ids_hbm.at[base + j], idx[j])
            pltpu.async_copy(word_hbm.at[idx[j]], rows[j], gsem[j])

        def do_add(parity, rows_b):
            start = parity * CHUNK

            def add_row(i, carry):
                p = start + i
                for k in range(NCOL):
                    sl = pl.ds(k * NLANE, NLANE)
                    plsc.addupdate(rows_b.at[i, sl], pos_v[p, sl])
                return carry

            lax.fori_loop(0, CHUNK, add_row, 0)

        def body(g, b):
            b2 = (b + LOOK) % NBUF
            c = base + g

            @pl.when(g + LOOK < n)
            def _():
                pltpu.async_copy(ids_hbm.at[c + LOOK], idx[b2], isem[b2])

            pltpu.make_async_copy(word_hbm.at[idx[b]], rows[b], gsem[b]).wait()

            @pl.when(g + LOOK < n)
            def _():
                @pl.when(g >= NBUF - LOOK)
                def _():
                    pltpu.make_async_copy(rows[b2],
                                          out_hbm.at[c + LOOK - NBUF],
                                          ssem[b2]).wait()

                pltpu.make_async_copy(ids_hbm.at[c + LOOK], idx[b2],
                                      isem[b2]).wait()
                pltpu.async_copy(word_hbm.at[idx[b2]], rows[b2], gsem[b2])

            do_add(b % 2, rows[b])
            pltpu.async_copy(rows[b], out_hbm.at[c], ssem[b])

        def quad(t, carry):
            for b in range(NBUF):
                body(NBUF * t + b, b)
            return carry

        lax.fori_loop(0, n // NBUF, quad, 0)

        for j in range(NBUF):
            g = n - NBUF + j
            pltpu.make_async_copy(rows[g % NBUF], out_hbm.at[base + g],
                                  ssem[g % NBUF]).wait()

    return emb


_emb_kernel = _build(32)


def kernel(input_ids, word_emb, pos_emb):
    ids = input_ids.reshape(NCHUNKS, CHUNK).astype(jnp.int32)
    out = _emb_kernel(ids, word_emb, pos_emb)
    return out.reshape(B, S, HIDDEN)


# speedup vs baseline: 17.1199x; 2.2364x over previous
import functools

import jax
import jax.numpy as jnp
from jax import lax
from jax.experimental import pallas as pl
from jax.experimental.pallas import tpu as pltpu
from jax.experimental.pallas import tpu_sc as plsc

VOCAB = 100000
HIDDEN = 128
B = 4096
S = 200
HALF = 100
NLANE = 16
NCOL = HIDDEN // NLANE
NBUF = 3


def _build(num_workers):
    n = B // num_workers
    mesh = plsc.VectorSubcoreMesh(core_axis_name="c", subcore_axis_name="s")

    @functools.partial(
        pl.kernel,
        mesh=mesh,
        out_type=jax.ShapeDtypeStruct((B, S, HIDDEN), jnp.float32),
        scratch_types=(
            [pltpu.VMEM((2, HALF), jnp.int32) for _ in range(NBUF)]
            + [pltpu.VMEM((S, HIDDEN), jnp.float32) for _ in range(NBUF)]
            + [pltpu.VMEM((S, HIDDEN), jnp.float32)]
            + [pltpu.SemaphoreType.DMA for _ in range(3 * NBUF)]
        ),
    )
    def emb(ids_hbm, word_hbm, pos_hbm, out_hbm, *scratch):
        idx = scratch[:NBUF]
        rows = scratch[NBUF:2 * NBUF]
        pos_v = scratch[2 * NBUF]
        gsem = scratch[2 * NBUF + 1:2 * NBUF + 1 + NBUF]
        ssem = scratch[2 * NBUF + 1 + NBUF:2 * NBUF + 1 + 2 * NBUF]
        isem = scratch[2 * NBUF + 1 + 2 * NBUF:]

        cid = lax.axis_index("c")
        sid = lax.axis_index("s")
        wid = sid * 2 + cid
        base = wid * n

        pltpu.sync_copy(pos_hbm.at[pl.ds(0, S)], pos_v)

        def start_gathers(bb, c):
            for h in range(2):
                pltpu.async_copy(word_hbm.at[idx[bb].at[h]],
                                 rows[bb].at[pl.ds(h * HALF, HALF)],
                                 gsem[bb])

        def wait_gathers(bb, c):
            for h in range(2):
                pltpu.make_async_copy(word_hbm.at[idx[bb].at[h]],
                                      rows[bb].at[pl.ds(h * HALF, HALF)],
                                      gsem[bb]).wait()

        pltpu.sync_copy(ids_hbm.at[base], idx[0])
        start_gathers(0, base)

        def do_add(rows_b):
            def add_row(i, carry):
                for k in range(NCOL):
                    sl = pl.ds(k * NLANE, NLANE)
                    plsc.addupdate(rows_b.at[i, sl], pos_v[i, sl])
                return carry

            lax.fori_loop(0, S, add_row, 0)

        def when(pred, fn):
            if isinstance(pred, bool):
                if pred:
                    fn()
            else:
                pl.when(pred)(fn)

        def body(g, b):
            nb = (b + 1) % NBUF
            c = base + g
            has_next = g + 1 < n

            def start_idx_copy():
                pltpu.async_copy(ids_hbm.at[c + 1], idx[nb], isem[nb])

            when(has_next, start_idx_copy)

            wait_gathers(b, c)

            def prefetch():
                def drain_scatter():
                    pltpu.make_async_copy(rows[nb], out_hbm.at[c + 1 - NBUF],
                                          ssem[nb]).wait()

                when(g >= NBUF - 1, drain_scatter)
                pltpu.make_async_copy(ids_hbm.at[c + 1], idx[nb],
                                      isem[nb]).wait()
                start_gathers(nb, c + 1)

            when(has_next, prefetch)

            do_add(rows[b])
            pltpu.async_copy(rows[b], out_hbm.at[c], ssem[b])

        def tri(t, carry):
            for b in range(NBUF):
                body(NBUF * t + b, b)
            return carry

        ntri = n // NBUF
        lax.fori_loop(0, ntri, tri, 0)
        for j in range(ntri * NBUF, n):
            body(j, j % NBUF)

        for g in range(n - NBUF, n):
            pltpu.make_async_copy(rows[g % NBUF], out_hbm.at[base + g],
                                  ssem[g % NBUF]).wait()

    return emb


_emb_kernel = _build(32)


def kernel(input_ids, word_emb, pos_emb):
    ids = input_ids.reshape(B, 2, HALF).astype(jnp.int32)
    return _emb_kernel(ids, word_emb, pos_emb)


# speedup vs baseline: 17.1968x; 1.0045x over previous
import functools

import jax
import jax.numpy as jnp
from jax import lax
from jax.experimental import pallas as pl
from jax.experimental.pallas import tpu as pltpu
from jax.experimental.pallas import tpu_sc as plsc

VOCAB = 100000
HIDDEN = 128
B = 4096
S = 200
HALF = 100
NLANE = 16
NCOL = HIDDEN // NLANE
NBUF = 3


def _build(num_workers):
    n = B // num_workers
    mesh = plsc.VectorSubcoreMesh(core_axis_name="c", subcore_axis_name="s")

    @functools.partial(
        pl.kernel,
        mesh=mesh,
        out_type=jax.ShapeDtypeStruct((B, S, HIDDEN), jnp.float32),
        scratch_types=(
            [pltpu.VMEM((S,), jnp.int32) for _ in range(NBUF)]
            + [pltpu.VMEM((S, HIDDEN), jnp.float32) for _ in range(NBUF)]
            + [pltpu.VMEM((S, HIDDEN), jnp.float32)]
            + [pltpu.SemaphoreType.DMA for _ in range(3 * NBUF)]
        ),
    )
    def emb(ids_hbm, word_hbm, pos_hbm, out_hbm, *scratch):
        idx = scratch[:NBUF]
        rows = scratch[NBUF:2 * NBUF]
        pos_v = scratch[2 * NBUF]
        gsem = scratch[2 * NBUF + 1:2 * NBUF + 1 + NBUF]
        ssem = scratch[2 * NBUF + 1 + NBUF:2 * NBUF + 1 + 2 * NBUF]
        isem = scratch[2 * NBUF + 1 + 2 * NBUF:]

        cid = lax.axis_index("c")
        sid = lax.axis_index("s")
        wid = sid * 2 + cid
        base = wid * n

        pltpu.sync_copy(pos_hbm.at[pl.ds(0, S)], pos_v)

        splits = ((0, 128), (128, 72))

        def start_gathers(bb, c):
            for off, ln in splits:
                pltpu.async_copy(word_hbm.at[idx[bb].at[pl.ds(off, ln)]],
                                 rows[bb].at[pl.ds(off, ln)],
                                 gsem[bb])

        def wait_gathers(bb, c):
            for off, ln in splits:
                pltpu.make_async_copy(
                    word_hbm.at[idx[bb].at[pl.ds(off, ln)]],
                    rows[bb].at[pl.ds(off, ln)],
                    gsem[bb]).wait()

        pltpu.sync_copy(ids_hbm.at[base], idx[0])
        start_gathers(0, base)

        def do_add(rows_b):
            def add_row(i, carry):
                for k in range(NCOL):
                    sl = pl.ds(k * NLANE, NLANE)
                    plsc.addupdate(rows_b.at[i, sl], pos_v[i, sl])
                return carry

            lax.fori_loop(0, S, add_row, 0)

        def when(pred, fn):
            if isinstance(pred, bool):
                if pred:
                    fn()
            else:
                pl.when(pred)(fn)

        def body(g, b):
            nb = (b + 1) % NBUF
            c = base + g
            has_next = g + 1 < n

            def start_idx_copy():
                pltpu.async_copy(ids_hbm.at[c + 1], idx[nb], isem[nb])

            when(has_next, start_idx_copy)

            wait_gathers(b, c)

            def prefetch():
                def drain_scatter():
                    pltpu.make_async_copy(rows[nb], out_hbm.at[c + 1 - NBUF],
                                          ssem[nb]).wait()

                when(g >= NBUF - 1, drain_scatter)
                pltpu.make_async_copy(ids_hbm.at[c + 1], idx[nb],
                                      isem[nb]).wait()
                start_gathers(nb, c + 1)

            when(has_next, prefetch)

            do_add(rows[b])
            pltpu.async_copy(rows[b], out_hbm.at[c], ssem[b])

        def tri(t, carry):
            for b in range(NBUF):
                body(NBUF * t + b, b)
            return carry

        ntri = n // NBUF
        lax.fori_loop(0, ntri, tri, 0)
        for j in range(ntri * NBUF, n):
            body(j, j % NBUF)

        for g in range(n - NBUF, n):
            pltpu.make_async_copy(rows[g % NBUF], out_hbm.at[base + g],
                                  ssem[g % NBUF]).wait()

    return emb


_emb_kernel = _build(32)


def kernel(input_ids, word_emb, pos_emb):
    return _emb_kernel(input_ids.astype(jnp.int32), word_emb, pos_emb)
